# baseline (device time: 199990 ns/iter reference)
import jax
import jax.numpy as jnp
from jax import lax
from jax.experimental import pallas as pl
from jax.experimental.pallas import tpu as pltpu

M = 8192
N = 2048
HALF = 1024
NCHUNK = 16
RC = M // NCHUNK


def kernel(x):
    def body(x_hbm, out_hbm, send_buf, recv_buf, acc, stage_a,
             stage_b, copy_sems_a, copy_sems_b, send_sems, recv_sems,
             out_sems):
        my_x = lax.axis_index("x")
        my_y = lax.axis_index("y")
        my_z = lax.axis_index("z")
        peer = (my_x, 1 - my_y, my_z)

        my_col = my_y * HALF
        peer_col = (1 - my_y) * HALF

        def load(c, col, stage, sems):
            return pltpu.make_async_copy(
                x_hbm.at[0, pl.ds(c * RC, RC), pl.ds(col, HALF)],
                stage.at[c % 2],
                sems.at[c % 2],
            )

        def chunk_rdma(c):
            rows = pl.ds(c * RC, RC)
            return pltpu.make_async_remote_copy(
                src_ref=send_buf.at[rows, :],
                dst_ref=recv_buf.at[rows, :],
                send_sem=send_sems.at[c],
                recv_sem=recv_sems.at[c],
                device_id=peer,
                device_id_type=pl.DeviceIdType.MESH,
            )

        def store_out(c):
            rows = pl.ds(c * RC, RC)
            return pltpu.make_async_copy(
                acc.at[c % 2], out_hbm.at[rows, :], out_sems.at[c]
            )

        barrier_sem = pltpu.get_barrier_semaphore()
        pl.semaphore_signal(barrier_sem, inc=1, device_id=peer,
                            device_id_type=pl.DeviceIdType.MESH)
        load(0, peer_col, stage_a, copy_sems_a).start()

        for c in range(NCHUNK):
            rows = pl.ds(c * RC, RC)
            load(c, peer_col, stage_a, copy_sems_a).wait()
            if c + 1 < NCHUNK:
                load(c + 1, peer_col, stage_a, copy_sems_a).start()
            send_buf[rows, :] = stage_a[c % 2].astype(jnp.bfloat16)
            if c == 0:
                pl.semaphore_wait(barrier_sem, 1)
            chunk_rdma(c).start()

        load(0, my_col, stage_b, copy_sems_b).start()
        load(1, my_col, stage_b, copy_sems_b).start()
        for c in range(NCHUNK):
            rows = pl.ds(c * RC, RC)
            load(c, my_col, stage_b, copy_sems_b).wait()
            chunk_rdma(c).wait_recv()
            if c >= 2:
                store_out(c - 2).wait()
            acc[c % 2] = recv_buf[rows, :] + stage_b[c % 2].astype(
                jnp.bfloat16)
            store_out(c).start()
            if c + 2 < NCHUNK:
                load(c + 2, my_col, stage_b, copy_sems_b).start()

        for c in range(NCHUNK - 2, NCHUNK):
            store_out(c).wait()
        for c in range(NCHUNK):
            chunk_rdma(c).wait_send()

    out_shape = jax.ShapeDtypeStruct((M, HALF), jnp.bfloat16)
    return pl.pallas_call(
        body,
        out_shape=out_shape,
        in_specs=[pl.BlockSpec(memory_space=pl.ANY)],
        out_specs=pl.BlockSpec(memory_space=pl.ANY),
        scratch_shapes=[
            pltpu.VMEM((M, HALF), jnp.bfloat16),
            pltpu.VMEM((M, HALF), jnp.bfloat16),
            pltpu.VMEM((2, RC, HALF), jnp.bfloat16),
            pltpu.VMEM((2, RC, HALF), jnp.float32),
            pltpu.VMEM((2, RC, HALF), jnp.float32),
            pltpu.SemaphoreType.DMA((2,)),
            pltpu.SemaphoreType.DMA((2,)),
            pltpu.SemaphoreType.DMA((NCHUNK,)),
            pltpu.SemaphoreType.DMA((NCHUNK,)),
            pltpu.SemaphoreType.DMA((NCHUNK,)),
        ],
        compiler_params=pltpu.CompilerParams(
            collective_id=0, vmem_limit_bytes=100 * 1024 * 1024
        ),
    )(x)
